# Initial kernel scaffold; baseline (speedup 1.0000x reference)
import jax
import jax.numpy as jnp
from jax.experimental import pallas as pl


def kernel(tokens, table):
    raise NotImplementedError("write your pallas kernel here")


# speedup vs baseline: 1.0148x; 1.0148x over previous
import functools
import math

import jax
import jax.numpy as jnp
from jax import lax
from jax.experimental import pallas as pl
from jax.experimental.pallas import tpu as pltpu
from jax.experimental.pallas import tpu_sc as plsc

EMB = 32
LANES = 16
IDXW = 128
G = 8
NC = 2
NS = 16
NW = NC * NS
SCALE = float(math.sqrt(EMB))


def _make_emb(total: int):
    nrows = total // IDXW
    rows_per_w = nrows // NW
    steps = rows_per_w // G
    chunk = G * IDXW

    mesh = plsc.VectorSubcoreMesh(core_axis_name="c", subcore_axis_name="s")

    @functools.partial(
        pl.kernel,
        mesh=mesh,
        out_type=jax.ShapeDtypeStruct((total, EMB), jnp.float32),
        scratch_types=[
            pltpu.VMEM((G, IDXW), jnp.int32),
            pltpu.VMEM((chunk, EMB), jnp.float32),
            pltpu.SemaphoreType.DMA,
        ],
        compiler_params=pltpu.CompilerParams(use_tc_tiling_on_sc=False),
    )
    def emb(tok_ref, table_ref, out_ref, idx_v, rows_v, sem):
        wid = lax.axis_index("s") * NC + lax.axis_index("c")
        row_base = wid * rows_per_w

        def step(s, carry):
            rb = row_base + s * G
            pltpu.sync_copy(tok_ref.at[pl.ds(rb, G)], idx_v)
            copies = [
                pltpu.async_copy(
                    table_ref.at[idx_v.at[j]],
                    rows_v.at[pl.ds(j * IDXW, IDXW)],
                    sem,
                )
                for j in range(G)
            ]
            for c in copies:
                c.wait()

            def scale_body(i, c2):
                for r in range(4):
                    row = i * 4 + r
                    for h in range(EMB // LANES):
                        sl = pl.ds(h * LANES, LANES)
                        rows_v[row, sl] = rows_v[row, sl] * SCALE
                return c2

            lax.fori_loop(0, chunk // 4, scale_body, 0, unroll=2)
            pltpu.sync_copy(rows_v, out_ref.at[pl.ds(rb * IDXW, chunk)])
            return carry

        lax.fori_loop(0, steps, step, 0)

    return emb


def kernel(tokens, table):
    b, l = tokens.shape
    total = b * l
    tok2d = tokens.reshape(total // IDXW, IDXW)
    out = _make_emb(total)(tok2d, table)
    return out.reshape(b, l, table.shape[1])


# speedup vs baseline: 1.5984x; 1.5750x over previous
import functools
import math

import jax
import jax.numpy as jnp
from jax import lax
from jax.experimental import pallas as pl
from jax.experimental.pallas import tpu as pltpu
from jax.experimental.pallas import tpu_sc as plsc

EMB = 32
LANES = 16
G = 16
NC = 2
NS = 16
NW = NC * NS
SCALE = float(math.sqrt(EMB))


def _make_emb(b: int, l: int):
    rows_per_w = b // NW
    steps = rows_per_w // G

    mesh = plsc.VectorSubcoreMesh(core_axis_name="c", subcore_axis_name="s")

    @functools.partial(
        pl.kernel,
        mesh=mesh,
        out_type=jax.ShapeDtypeStruct((b, l, EMB), jnp.float32),
        scratch_types=[
            pltpu.VMEM((G, l), jnp.int32),
            pltpu.VMEM((G, l, EMB), jnp.float32),
            pltpu.SemaphoreType.DMA,
        ],
        compiler_params=pltpu.CompilerParams(use_tc_tiling_on_sc=False),
    )
    def emb(tok_ref, table_ref, out_ref, idx_v, rows_v, sem):
        wid = lax.axis_index("s") * NC + lax.axis_index("c")
        row_base = wid * rows_per_w

        def step(s, carry):
            rb = row_base + s * G
            pltpu.sync_copy(tok_ref.at[pl.ds(rb, G)], idx_v)
            copies = [
                pltpu.async_copy(table_ref.at[idx_v.at[j]], rows_v.at[j], sem)
                for j in range(G)
            ]
            for c in copies:
                c.wait()

            def scale_body(t, c2):
                for j in range(G):
                    for h in range(EMB // LANES):
                        sl = pl.ds(h * LANES, LANES)
                        rows_v[j, t, sl] = rows_v[j, t, sl] * SCALE
                return c2

            lax.fori_loop(0, l, scale_body, 0)
            pltpu.sync_copy(rows_v, out_ref.at[pl.ds(rb, G)])
            return carry

        lax.fori_loop(0, steps, step, 0)

    return emb


def kernel(tokens, table):
    b, l = tokens.shape
    return _make_emb(b, l)(tokens, table)
